# baseline (speedup 1.0000x reference)
import functools

import jax
import jax.numpy as jnp
from jax import lax
from jax.experimental import pallas as pl
from jax.experimental.pallas import tpu as pltpu
from jax.experimental.pallas import tpu_sc as plsc

B, C, H, W, NREF = 2, 192, 224, 224, 4
HW = H * W
BHW = B * HW
CP = 192


BLK = 1024
N_BLK = HW // BLK


def _proj_body(x_ref, wq_ref, bq_ref, wk_ref, bk_ref, wv_ref, bv_ref,
               wo_ref, bo_ref, qt_ref, kt_ref, vt_ref, idx_ref):
    b = pl.program_id(0)
    j = pl.program_id(1)
    xb = x_ref[0]
    cdims = (((0,), (1,)), ((), ()))
    qt = lax.dot_general(xb, wq_ref[...], cdims,
                         preferred_element_type=jnp.float32) + bq_ref[...]
    qt_ref[...] = qt
    kt_ref[...] = lax.dot_general(xb, wk_ref[...], cdims,
                                  preferred_element_type=jnp.float32) + bk_ref[...]
    vt_ref[...] = lax.dot_general(xb, wv_ref[...], cdims,
                                  preferred_element_type=jnp.float32) + bv_ref[...]
    offs = lax.dot_general(wo_ref[...], qt, (((1,), (1,)), ((), ())),
                           preferred_element_type=jnp.float32) + bo_ref[...]
    p = j * BLK + lax.broadcasted_iota(jnp.int32, (1, BLK), 1)
    hpos = (p // W).astype(jnp.float32)
    wpos = (p % W).astype(jnp.float32)
    offs = offs.reshape(NREF, 2, BLK)
    ref_w = jnp.clip(wpos + offs[:, 0, :], 0.0, float(W - 1)).astype(jnp.int32)
    ref_h = jnp.clip(hpos + offs[:, 1, :], 0.0, float(H - 1)).astype(jnp.int32)
    idx_ref[...] = ref_h * W + ref_w + b * HW


def _stage1(x, Wq, bq, Wk, bk, Wv, bv, Wo, bo):
    xf = x.reshape(B, C, HW)
    grid = (B, N_BLK)
    wspec = pl.BlockSpec((C, C), lambda b, j: (0, 0))
    rspec = pl.BlockSpec((1, C), lambda b, j: (0, 0))
    return pl.pallas_call(
        _proj_body,
        grid=grid,
        in_specs=[
            pl.BlockSpec((1, C, BLK), lambda b, j: (b, 0, j)),
            wspec, rspec,
            wspec, rspec,
            wspec, rspec,
            pl.BlockSpec((2 * NREF, C), lambda b, j: (0, 0)),
            pl.BlockSpec((2 * NREF, 1), lambda b, j: (0, 0)),
        ],
        out_specs=[
            pl.BlockSpec((BLK, C), lambda b, j: (b * N_BLK + j, 0)),
            pl.BlockSpec((BLK, CP), lambda b, j: (b * N_BLK + j, 0)),
            pl.BlockSpec((BLK, CP), lambda b, j: (b * N_BLK + j, 0)),
            pl.BlockSpec((NREF, BLK), lambda b, j: (0, b * N_BLK + j)),
        ],
        out_shape=[
            jax.ShapeDtypeStruct((BHW, C), jnp.float32),
            jax.ShapeDtypeStruct((BHW, CP), jnp.float32),
            jax.ShapeDtypeStruct((BHW, CP), jnp.float32),
            jax.ShapeDtypeStruct((NREF, BHW), jnp.int32),
        ],
    )(xf, Wq, bq.reshape(1, C), Wk, bk.reshape(1, C), Wv, bv.reshape(1, C),
      Wo, bo.reshape(2 * NREF, 1))


NC, NS, L = 2, 16, 16
NW = NC * NS
P = 128
SUB = 16
NSUB = P // SUB
NCHUNK = BHW // P
NJ = C // L


def _sc_body(qt_hbm, kt_hbm, vt_hbm, idx_hbm, out_hbm,
             idxv, kg, vg, qv, ov, sem0, sem1):
    wid = lax.axis_index("s") * NC + lax.axis_index("c")
    nchunks = jnp.where(wid < NCHUNK % NW, NCHUNK // NW + 1, NCHUNK // NW)
    sems = (sem0, sem1)

    def chunk_body(t, _):
        ci = wid + t * NW
        base = pl.multiple_of(ci * P, P)
        pltpu.sync_copy(idx_hbm.at[:, pl.ds(base, P)], idxv)
        pltpu.sync_copy(qt_hbm.at[pl.ds(base, P)], qv)

        def fire(sub):
            slot = sub % 2
            cps = []
            for r in range(NREF):
                ixr = idxv.at[r, pl.ds(sub * SUB, SUB)]
                cps.append(pltpu.async_copy(kt_hbm.at[ixr], kg.at[slot, r],
                                            sems[slot]))
                cps.append(pltpu.async_copy(vt_hbm.at[ixr], vg.at[slot, r],
                                            sems[slot]))
            return cps

        pend = fire(0)
        for sub in range(NSUB):
            nxt = fire(sub + 1) if sub + 1 < NSUB else None
            for cp in pend:
                cp.wait()
            slot = sub % 2

            @plsc.parallel_loop(0, SUB, 1, unroll=2)
            def pos_body(i):
                pi = sub * SUB + i
                qvecs = []
                for jgrp in range(NJ):
                    qvecs.append(qv[pi, pl.ds(jgrp * L, L)])
                s = []
                for r in range(NREF):
                    acc = qvecs[0] * kg[slot, r, i, pl.ds(0, L)]
                    for jgrp in range(1, NJ):
                        acc = acc + qvecs[jgrp] * kg[slot, r, i, pl.ds(jgrp * L, L)]
                    s.append(jnp.sum(acc))
                for jgrp in range(NJ):
                    o = s[0] * vg[slot, 0, i, pl.ds(jgrp * L, L)]
                    for r in range(1, NREF):
                        o = o + s[r] * vg[slot, r, i, pl.ds(jgrp * L, L)]
                    ov[pi, pl.ds(jgrp * L, L)] = o

            pend = nxt

        pltpu.sync_copy(ov, out_hbm.at[pl.ds(base, P)])
        return ()

    lax.fori_loop(0, nchunks, chunk_body, (), unroll=False)


@functools.cache
def _sc_attn():
    return pl.kernel(
        _sc_body,
        out_type=jax.ShapeDtypeStruct((BHW, C), jnp.float32),
        mesh=plsc.VectorSubcoreMesh(core_axis_name="c", subcore_axis_name="s",
                                    num_cores=NC, num_subcores=NS),
        compiler_params=pltpu.CompilerParams(needs_layout_passes=False,
                                             use_tc_tiling_on_sc=False),
        scratch_types=[
            pltpu.VMEM((NREF, P), jnp.int32),
            pltpu.VMEM((2, NREF, SUB, CP), jnp.float32),
            pltpu.VMEM((2, NREF, SUB, CP), jnp.float32),
            pltpu.VMEM((P, C), jnp.float32),
            pltpu.VMEM((P, C), jnp.float32),
            pltpu.SemaphoreType.DMA,
            pltpu.SemaphoreType.DMA,
        ],
    )


TBLK = 512
NT_BLK = HW // TBLK


def _tr_body(ot_ref, o_ref):
    o_ref[0] = ot_ref[...].T


def _stage3(outT):
    return pl.pallas_call(
        _tr_body,
        grid=(B, NT_BLK),
        in_specs=[pl.BlockSpec((TBLK, C), lambda b, j: (b * NT_BLK + j, 0))],
        out_specs=pl.BlockSpec((1, C, TBLK), lambda b, j: (b, 0, j)),
        out_shape=jax.ShapeDtypeStruct((B, C, HW), jnp.float32),
    )(outT)


def kernel(x, Wq, bq, Wk, bk, Wv, bv, Wo, bo):
    qt, kt, vt, idx = _stage1(x, Wq, bq, Wk, bk, Wv, bv, Wo, bo)
    outT = _sc_attn()(qt, kt, vt, idx)
    return _stage3(outT).reshape(B, C, H, W)


# speedup vs baseline: 1.2518x; 1.2518x over previous
import functools

import jax
import jax.numpy as jnp
from jax import lax
from jax.experimental import pallas as pl
from jax.experimental.pallas import tpu as pltpu
from jax.experimental.pallas import tpu_sc as plsc

B, C, H, W, NREF = 2, 192, 224, 224, 4
HW = H * W
BHW = B * HW
CP = 256


BLK = 1024
N_BLK = HW // BLK


def _proj_body(x_ref, wq_ref, bq_ref, wk_ref, bk_ref, wv_ref, bv_ref,
               wo_ref, bo_ref, qt_ref, kt_ref, vt_ref, idx_ref):
    b = pl.program_id(0)
    j = pl.program_id(1)
    xb = x_ref[0]
    cdims = (((0,), (1,)), ((), ()))
    pad = jnp.zeros((BLK, CP - C), jnp.float32)
    qt = lax.dot_general(xb, wq_ref[...], cdims,
                         preferred_element_type=jnp.float32) + bq_ref[...]
    qt_ref[...] = qt
    kt = lax.dot_general(xb, wk_ref[...], cdims,
                         preferred_element_type=jnp.float32) + bk_ref[...]
    kt_ref[...] = jnp.concatenate([kt, pad], axis=1)
    vt = lax.dot_general(xb, wv_ref[...], cdims,
                         preferred_element_type=jnp.float32) + bv_ref[...]
    vt_ref[...] = jnp.concatenate([vt, pad], axis=1)
    offs = lax.dot_general(wo_ref[...], qt, (((1,), (1,)), ((), ())),
                           preferred_element_type=jnp.float32) + bo_ref[...]
    p = j * BLK + lax.broadcasted_iota(jnp.int32, (1, BLK), 1)
    hpos = (p // W).astype(jnp.float32)
    wpos = (p % W).astype(jnp.float32)
    offs = offs.reshape(NREF, 2, BLK)
    ref_w = jnp.clip(wpos + offs[:, 0, :], 0.0, float(W - 1)).astype(jnp.int32)
    ref_h = jnp.clip(hpos + offs[:, 1, :], 0.0, float(H - 1)).astype(jnp.int32)
    idx_ref[...] = ref_h * W + ref_w + b * HW


def _stage1(x, Wq, bq, Wk, bk, Wv, bv, Wo, bo):
    xf = x.reshape(B, C, HW)
    grid = (B, N_BLK)
    wspec = pl.BlockSpec((C, C), lambda b, j: (0, 0))
    rspec = pl.BlockSpec((1, C), lambda b, j: (0, 0))
    return pl.pallas_call(
        _proj_body,
        grid=grid,
        in_specs=[
            pl.BlockSpec((1, C, BLK), lambda b, j: (b, 0, j)),
            wspec, rspec,
            wspec, rspec,
            wspec, rspec,
            pl.BlockSpec((2 * NREF, C), lambda b, j: (0, 0)),
            pl.BlockSpec((2 * NREF, 1), lambda b, j: (0, 0)),
        ],
        out_specs=[
            pl.BlockSpec((BLK, C), lambda b, j: (b * N_BLK + j, 0)),
            pl.BlockSpec((BLK, CP), lambda b, j: (b * N_BLK + j, 0)),
            pl.BlockSpec((BLK, CP), lambda b, j: (b * N_BLK + j, 0)),
            pl.BlockSpec((NREF, BLK), lambda b, j: (0, b * N_BLK + j)),
        ],
        out_shape=[
            jax.ShapeDtypeStruct((BHW, C), jnp.float32),
            jax.ShapeDtypeStruct((BHW, CP), jnp.float32),
            jax.ShapeDtypeStruct((BHW, CP), jnp.float32),
            jax.ShapeDtypeStruct((NREF, BHW), jnp.int32),
        ],
    )(xf, Wq, bq.reshape(1, C), Wk, bk.reshape(1, C), Wv, bv.reshape(1, C),
      Wo, bo.reshape(2 * NREF, 1))


NC, NS, L = 2, 16, 16
NW = NC * NS
P = 128
SUB = 8
NSUB = P // SUB
NCHUNK = BHW // P
NJ = C // L


def _sc_body(qt_hbm, kt_hbm, vt_hbm, idx_hbm, out_hbm,
             idxv, kg, vg, qv, ov, sem0, sem1):
    wid = lax.axis_index("s") * NC + lax.axis_index("c")
    nchunks = jnp.where(wid < NCHUNK % NW, NCHUNK // NW + 1, NCHUNK // NW)
    sems = (sem0, sem1)

    def chunk_body(t, _):
        ci = wid + t * NW
        base = pl.multiple_of(ci * P, P)
        pltpu.sync_copy(idx_hbm.at[:, pl.ds(base, P)], idxv)
        pltpu.sync_copy(qt_hbm.at[pl.ds(base, P)], qv)

        def fire(sub):
            slot = sub % 2
            cps = []
            for r in range(NREF):
                ixr = idxv.at[r, pl.ds(sub * SUB, SUB)]
                cps.append(pltpu.async_copy(kt_hbm.at[ixr], kg.at[slot, r],
                                            sems[slot]))
                cps.append(pltpu.async_copy(vt_hbm.at[ixr], vg.at[slot, r],
                                            sems[slot]))
            return cps

        pend = fire(0)
        for sub in range(NSUB):
            nxt = fire(sub + 1) if sub + 1 < NSUB else None
            for cp in pend:
                cp.wait()
            slot = sub % 2

            @plsc.parallel_loop(0, SUB, 1, unroll=2)
            def pos_body(i):
                pi = sub * SUB + i
                qvecs = []
                for jgrp in range(NJ):
                    qvecs.append(qv[pi, pl.ds(jgrp * L, L)])
                s = []
                for r in range(NREF):
                    acc = qvecs[0] * kg[slot, r, i, pl.ds(0, L)]
                    for jgrp in range(1, NJ):
                        acc = acc + qvecs[jgrp] * kg[slot, r, i, pl.ds(jgrp * L, L)]
                    s.append(jnp.sum(acc))
                for jgrp in range(NJ):
                    o = s[0] * vg[slot, 0, i, pl.ds(jgrp * L, L)]
                    for r in range(1, NREF):
                        o = o + s[r] * vg[slot, r, i, pl.ds(jgrp * L, L)]
                    ov[pi, pl.ds(jgrp * L, L)] = o

            pend = nxt

        pltpu.sync_copy(ov, out_hbm.at[pl.ds(base, P)])
        return ()

    lax.fori_loop(0, nchunks, chunk_body, (), unroll=False)


@functools.cache
def _sc_attn():
    return pl.kernel(
        _sc_body,
        out_type=jax.ShapeDtypeStruct((BHW, C), jnp.float32),
        mesh=plsc.VectorSubcoreMesh(core_axis_name="c", subcore_axis_name="s",
                                    num_cores=NC, num_subcores=NS),
        compiler_params=pltpu.CompilerParams(needs_layout_passes=False),
        scratch_types=[
            pltpu.VMEM((NREF, P), jnp.int32),
            pltpu.VMEM((2, NREF, SUB, CP), jnp.float32),
            pltpu.VMEM((2, NREF, SUB, CP), jnp.float32),
            pltpu.VMEM((P, C), jnp.float32),
            pltpu.VMEM((P, C), jnp.float32),
            pltpu.SemaphoreType.DMA,
            pltpu.SemaphoreType.DMA,
        ],
    )


TBLK = 512
NT_BLK = HW // TBLK


def _tr_body(ot_ref, o_ref):
    o_ref[0] = ot_ref[...].T


def _stage3(outT):
    return pl.pallas_call(
        _tr_body,
        grid=(B, NT_BLK),
        in_specs=[pl.BlockSpec((TBLK, C), lambda b, j: (b * NT_BLK + j, 0))],
        out_specs=pl.BlockSpec((1, C, TBLK), lambda b, j: (b, 0, j)),
        out_shape=jax.ShapeDtypeStruct((B, C, HW), jnp.float32),
    )(outT)


def kernel(x, Wq, bq, Wk, bk, Wv, bv, Wo, bo):
    qt, kt, vt, idx = _stage1(x, Wq, bq, Wk, bk, Wv, bv, Wo, bo)
    outT = _sc_attn()(qt, kt, vt, idx)
    return _stage3(outT).reshape(B, C, H, W)


# speedup vs baseline: 1.4234x; 1.1371x over previous
import functools

import jax
import jax.numpy as jnp
from jax import lax
from jax.experimental import pallas as pl
from jax.experimental.pallas import tpu as pltpu
from jax.experimental.pallas import tpu_sc as plsc

B, C, H, W, NREF = 2, 192, 224, 224, 4
HW = H * W
BHW = B * HW
CH = C // 2
CW = 128


BLK = 1024
N_BLK = HW // BLK


def _pack_tc(t):
    lo = lax.bitcast_convert_type(t[:, :CH].astype(jnp.bfloat16), jnp.uint16)
    hi = lax.bitcast_convert_type(t[:, CH:].astype(jnp.bfloat16), jnp.uint16)
    w = (hi.astype(jnp.uint32) << 16) | lo.astype(jnp.uint32)
    w = lax.bitcast_convert_type(w, jnp.float32)
    return jnp.concatenate([w, jnp.zeros((BLK, CW - CH), jnp.float32)], axis=1)


def _proj_body(x_ref, wq_ref, bq_ref, wk_ref, bk_ref, wv_ref, bv_ref,
               wo_ref, bo_ref, qt_ref, kt_ref, vt_ref, idx_ref):
    b = pl.program_id(0)
    j = pl.program_id(1)
    xb = x_ref[0]
    cdims = (((0,), (1,)), ((), ()))
    qt = lax.dot_general(xb, wq_ref[...], cdims,
                         preferred_element_type=jnp.float32) + bq_ref[...]
    qt_ref[...] = _pack_tc(qt)
    kt = lax.dot_general(xb, wk_ref[...], cdims,
                         preferred_element_type=jnp.float32) + bk_ref[...]
    kt_ref[...] = _pack_tc(kt)
    vt = lax.dot_general(xb, wv_ref[...], cdims,
                         preferred_element_type=jnp.float32) + bv_ref[...]
    vt_ref[...] = _pack_tc(vt)
    offs = lax.dot_general(wo_ref[...], qt, (((1,), (1,)), ((), ())),
                           preferred_element_type=jnp.float32) + bo_ref[...]
    p = j * BLK + lax.broadcasted_iota(jnp.int32, (1, BLK), 1)
    hpos = (p // W).astype(jnp.float32)
    wpos = (p % W).astype(jnp.float32)
    offs = offs.reshape(NREF, 2, BLK)
    ref_w = jnp.clip(wpos + offs[:, 0, :], 0.0, float(W - 1)).astype(jnp.int32)
    ref_h = jnp.clip(hpos + offs[:, 1, :], 0.0, float(H - 1)).astype(jnp.int32)
    idx_ref[...] = ref_h * W + ref_w + b * HW


def _stage1(x, Wq, bq, Wk, bk, Wv, bv, Wo, bo):
    xf = x.reshape(B, C, HW)
    grid = (B, N_BLK)
    wspec = pl.BlockSpec((C, C), lambda b, j: (0, 0))
    rspec = pl.BlockSpec((1, C), lambda b, j: (0, 0))
    return pl.pallas_call(
        _proj_body,
        grid=grid,
        in_specs=[
            pl.BlockSpec((1, C, BLK), lambda b, j: (b, 0, j)),
            wspec, rspec,
            wspec, rspec,
            wspec, rspec,
            pl.BlockSpec((2 * NREF, C), lambda b, j: (0, 0)),
            pl.BlockSpec((2 * NREF, 1), lambda b, j: (0, 0)),
        ],
        out_specs=[
            pl.BlockSpec((BLK, CW), lambda b, j: (b * N_BLK + j, 0)),
            pl.BlockSpec((BLK, CW), lambda b, j: (b * N_BLK + j, 0)),
            pl.BlockSpec((BLK, CW), lambda b, j: (b * N_BLK + j, 0)),
            pl.BlockSpec((NREF, BLK), lambda b, j: (0, b * N_BLK + j)),
        ],
        out_shape=[
            jax.ShapeDtypeStruct((BHW, CW), jnp.float32),
            jax.ShapeDtypeStruct((BHW, CW), jnp.float32),
            jax.ShapeDtypeStruct((BHW, CW), jnp.float32),
            jax.ShapeDtypeStruct((NREF, BHW), jnp.int32),
        ],
    )(xf, Wq, bq.reshape(1, C), Wk, bk.reshape(1, C), Wv, bv.reshape(1, C),
      Wo, bo.reshape(2 * NREF, 1))


NC, NS, L = 2, 16, 16
NW = NC * NS
P = 128
SUB = 32
NSUB = P // SUB
NCHUNK = BHW // P
NG = CH // L


def _unpack16(w16):
    return plsc.unpack(plsc.bitcast(w16, jnp.bfloat16),
                       format=plsc.PackFormat.INTERLEAVED)


def _sc_body(qt_hbm, kt_hbm, vt_hbm, idx_hbm, out_hbm,
             idxv, kg, vg, qv, ov, sem0, sem1):
    wid = lax.axis_index("s") * NC + lax.axis_index("c")
    nchunks = jnp.where(wid < NCHUNK % NW, NCHUNK // NW + 1, NCHUNK // NW)
    sems = (sem0, sem1)

    def chunk_body(t, _):
        ci = wid + t * NW
        base = pl.multiple_of(ci * P, P)
        pltpu.sync_copy(idx_hbm.at[:, pl.ds(base, P)], idxv)
        pltpu.sync_copy(qt_hbm.at[pl.ds(base, P)], qv)

        def fire(sub):
            slot = sub % 2
            cps = []
            for r in range(NREF):
                ixr = idxv.at[r, pl.ds(sub * SUB, SUB)]
                cps.append(pltpu.async_copy(kt_hbm.at[ixr], kg.at[slot, r],
                                            sems[slot]))
                cps.append(pltpu.async_copy(vt_hbm.at[ixr], vg.at[slot, r],
                                            sems[slot]))
            return cps

        pend = fire(0)
        for sub in range(NSUB):
            nxt = fire(sub + 1) if sub + 1 < NSUB else None
            for cp in pend:
                cp.wait()
            slot = sub % 2

            @plsc.parallel_loop(0, SUB, 1, unroll=2)
            def pos_body(i):
                pi = sub * SUB + i
                qa, qb = [], []
                for g in range(NG):
                    a, b_ = _unpack16(qv[pi, pl.ds(g * L, L)])
                    qa.append(a)
                    qb.append(b_)
                s = []
                for r in range(NREF):
                    acc = None
                    for g in range(NG):
                        ka, kb = _unpack16(kg[slot, r, i, pl.ds(g * L, L)])
                        term = ka * qa[g] + kb * qb[g]
                        acc = term if acc is None else acc + term
                    s.append(jnp.sum(acc))
                for g in range(NG):
                    va, vb = _unpack16(vg[slot, 0, i, pl.ds(g * L, L)])
                    oa = s[0] * va
                    ob = s[0] * vb
                    for r in range(1, NREF):
                        va, vb = _unpack16(vg[slot, r, i, pl.ds(g * L, L)])
                        oa = oa + s[r] * va
                        ob = ob + s[r] * vb
                    packed = plsc.pack(oa, ob, format=plsc.PackFormat.INTERLEAVED)
                    ov[pi, pl.ds(g * L, L)] = plsc.bitcast(packed, jnp.float32)

            pend = nxt

        pltpu.sync_copy(ov, out_hbm.at[pl.ds(base, P)])
        return ()

    lax.fori_loop(0, nchunks, chunk_body, (), unroll=False)


@functools.cache
def _sc_attn():
    return pl.kernel(
        _sc_body,
        out_type=jax.ShapeDtypeStruct((BHW, CW), jnp.float32),
        mesh=plsc.VectorSubcoreMesh(core_axis_name="c", subcore_axis_name="s",
                                    num_cores=NC, num_subcores=NS),
        compiler_params=pltpu.CompilerParams(needs_layout_passes=False),
        scratch_types=[
            pltpu.VMEM((NREF, P), jnp.int32),
            pltpu.VMEM((2, NREF, SUB, CW), jnp.float32),
            pltpu.VMEM((2, NREF, SUB, CW), jnp.float32),
            pltpu.VMEM((P, CW), jnp.float32),
            pltpu.VMEM((P, CW), jnp.float32),
            pltpu.SemaphoreType.DMA,
            pltpu.SemaphoreType.DMA,
        ],
    )


TBLK = 512
NT_BLK = HW // TBLK


def _tr_body(ot_ref, o_ref):
    u = lax.bitcast_convert_type(ot_ref[:, :CH], jnp.uint32)
    lo = lax.bitcast_convert_type((u & jnp.uint32(0xFFFF)).astype(jnp.uint16),
                                  jnp.bfloat16).astype(jnp.float32)
    hi = lax.bitcast_convert_type((u >> 16).astype(jnp.uint16),
                                  jnp.bfloat16).astype(jnp.float32)
    o_ref[0] = jnp.concatenate([lo, hi], axis=1).T


def _stage3(outP):
    return pl.pallas_call(
        _tr_body,
        grid=(B, NT_BLK),
        in_specs=[pl.BlockSpec((TBLK, CW), lambda b, j: (b * NT_BLK + j, 0))],
        out_specs=pl.BlockSpec((1, C, TBLK), lambda b, j: (b, 0, j)),
        out_shape=jax.ShapeDtypeStruct((B, C, HW), jnp.float32),
    )(outP)


def kernel(x, Wq, bq, Wk, bk, Wv, bv, Wo, bo):
    qt, kt, vt, idx = _stage1(x, Wq, bq, Wk, bk, Wv, bv, Wo, bo)
    outP = _sc_attn()(qt, kt, vt, idx)
    return _stage3(outP).reshape(B, C, H, W)


# speedup vs baseline: 1.5850x; 1.1135x over previous
import functools

import jax
import jax.numpy as jnp
from jax import lax
from jax.experimental import pallas as pl
from jax.experimental.pallas import tpu as pltpu
from jax.experimental.pallas import tpu_sc as plsc

B, C, H, W, NREF = 2, 192, 224, 224, 4
HW = H * W
CH = C // 2
CW = 128


BLK = 1024
N_BLK = HW // BLK


def _pack_tc(t):
    lo = lax.bitcast_convert_type(t[:, :CH].astype(jnp.bfloat16), jnp.uint16)
    hi = lax.bitcast_convert_type(t[:, CH:].astype(jnp.bfloat16), jnp.uint16)
    w = (hi.astype(jnp.uint32) << 16) | lo.astype(jnp.uint32)
    w = lax.bitcast_convert_type(w, jnp.float32)
    return jnp.concatenate([w, jnp.zeros((BLK, CW - CH), jnp.float32)], axis=1)


def _proj_body(x_ref, wq_ref, bq_ref, wk_ref, bk_ref, wv_ref, bv_ref,
               wo_ref, bo_ref, qt_ref, kt_ref, vt_ref, idx_ref):
    j = pl.program_id(0)
    xb = x_ref[...]
    xb_bf = xb.astype(jnp.bfloat16)
    cdims = (((0,), (1,)), ((), ()))
    qt = lax.dot_general(xb, wq_ref[...], cdims,
                         preferred_element_type=jnp.float32) + bq_ref[...]
    qt_ref[...] = _pack_tc(qt)
    kt = lax.dot_general(xb_bf, wk_ref[...].astype(jnp.bfloat16), cdims,
                         preferred_element_type=jnp.float32) + bk_ref[...]
    kt_ref[...] = _pack_tc(kt)
    vt = lax.dot_general(xb_bf, wv_ref[...].astype(jnp.bfloat16), cdims,
                         preferred_element_type=jnp.float32) + bv_ref[...]
    vt_ref[...] = _pack_tc(vt)
    offs = lax.dot_general(wo_ref[...], qt, (((1,), (1,)), ((), ())),
                           preferred_element_type=jnp.float32) + bo_ref[...]
    p = j * BLK + lax.broadcasted_iota(jnp.int32, (1, BLK), 1)
    hpos = (p // W).astype(jnp.float32)
    wpos = (p % W).astype(jnp.float32)
    offs = offs.reshape(NREF, 2, BLK)
    ref_w = jnp.clip(wpos + offs[:, 0, :], 0.0, float(W - 1)).astype(jnp.int32)
    ref_h = jnp.clip(hpos + offs[:, 1, :], 0.0, float(H - 1)).astype(jnp.int32)
    idx_ref[...] = ref_h * W + ref_w


def _stage1_half(xf, Wq, bq, Wk, bk, Wv, bv, Wo, bo):
    wspec = pl.BlockSpec((C, C), lambda j: (0, 0))
    rspec = pl.BlockSpec((1, C), lambda j: (0, 0))
    return pl.pallas_call(
        _proj_body,
        grid=(N_BLK,),
        in_specs=[
            pl.BlockSpec((C, BLK), lambda j: (0, j)),
            wspec, rspec,
            wspec, rspec,
            wspec, rspec,
            pl.BlockSpec((2 * NREF, C), lambda j: (0, 0)),
            pl.BlockSpec((2 * NREF, 1), lambda j: (0, 0)),
        ],
        out_specs=[
            pl.BlockSpec((BLK, CW), lambda j: (j, 0)),
            pl.BlockSpec((BLK, CW), lambda j: (j, 0)),
            pl.BlockSpec((BLK, CW), lambda j: (j, 0)),
            pl.BlockSpec((NREF, BLK), lambda j: (0, j)),
        ],
        out_shape=[
            jax.ShapeDtypeStruct((HW, CW), jnp.float32),
            jax.ShapeDtypeStruct((HW, CW), jnp.float32),
            jax.ShapeDtypeStruct((HW, CW), jnp.float32),
            jax.ShapeDtypeStruct((NREF, HW), jnp.int32),
        ],
    )(xf, Wq, bq.reshape(1, C), Wk, bk.reshape(1, C), Wv, bv.reshape(1, C),
      Wo, bo.reshape(2 * NREF, 1))


NC, NS, L = 2, 16, 16
NW = NC * NS
P = 128
SUB = 32
NSUB = P // SUB
NCHUNK = HW // P
NG = CH // L


def _unpack16(w16):
    return plsc.unpack(plsc.bitcast(w16, jnp.bfloat16),
                       format=plsc.PackFormat.INTERLEAVED)


def _sc_body(qt_hbm, kt_hbm, vt_hbm, idx_hbm, out_hbm,
             idxv, kg, vg, qv, ov, sem0, sem1):
    wid = lax.axis_index("s") * NC + lax.axis_index("c")
    nchunks = jnp.where(wid < NCHUNK % NW, NCHUNK // NW + 1, NCHUNK // NW)
    sems = (sem0, sem1)

    def chunk_body(t, _):
        ci = wid + t * NW
        base = pl.multiple_of(ci * P, P)
        pltpu.sync_copy(idx_hbm.at[:, pl.ds(base, P)], idxv)
        pltpu.sync_copy(qt_hbm.at[pl.ds(base, P)], qv)

        def fire(sub):
            slot = sub % 2
            cps = []
            for r in range(NREF):
                ixr = idxv.at[r, pl.ds(sub * SUB, SUB)]
                cps.append(pltpu.async_copy(kt_hbm.at[ixr], kg.at[slot, r],
                                            sems[slot]))
                cps.append(pltpu.async_copy(vt_hbm.at[ixr], vg.at[slot, r],
                                            sems[slot]))
            return cps

        pend = fire(0)
        for sub in range(NSUB):
            nxt = fire(sub + 1) if sub + 1 < NSUB else None
            for cp in pend:
                cp.wait()
            slot = sub % 2

            @plsc.parallel_loop(0, SUB, 1, unroll=2)
            def pos_body(i):
                pi = sub * SUB + i
                qa, qb = [], []
                for g in range(NG):
                    a, b_ = _unpack16(qv[pi, pl.ds(g * L, L)])
                    qa.append(a)
                    qb.append(b_)
                s = []
                for r in range(NREF):
                    acc = None
                    for g in range(NG):
                        ka, kb = _unpack16(kg[slot, r, i, pl.ds(g * L, L)])
                        term = ka * qa[g] + kb * qb[g]
                        acc = term if acc is None else acc + term
                    s.append(jnp.sum(acc))
                for g in range(NG):
                    va, vb = _unpack16(vg[slot, 0, i, pl.ds(g * L, L)])
                    oa = s[0] * va
                    ob = s[0] * vb
                    for r in range(1, NREF):
                        va, vb = _unpack16(vg[slot, r, i, pl.ds(g * L, L)])
                        oa = oa + s[r] * va
                        ob = ob + s[r] * vb
                    packed = plsc.pack(oa, ob, format=plsc.PackFormat.INTERLEAVED)
                    ov[pi, pl.ds(g * L, L)] = plsc.bitcast(packed, jnp.float32)

            pend = nxt

        pltpu.sync_copy(ov, out_hbm.at[pl.ds(base, P)])
        return ()

    lax.fori_loop(0, nchunks, chunk_body, (), unroll=False)


@functools.cache
def _sc_attn():
    return pl.kernel(
        _sc_body,
        out_type=jax.ShapeDtypeStruct((HW, CW), jnp.float32),
        mesh=plsc.VectorSubcoreMesh(core_axis_name="c", subcore_axis_name="s",
                                    num_cores=NC, num_subcores=NS),
        compiler_params=pltpu.CompilerParams(needs_layout_passes=False),
        scratch_types=[
            pltpu.VMEM((NREF, P), jnp.int32),
            pltpu.VMEM((2, NREF, SUB, CW), jnp.float32),
            pltpu.VMEM((2, NREF, SUB, CW), jnp.float32),
            pltpu.VMEM((P, CW), jnp.float32),
            pltpu.VMEM((P, CW), jnp.float32),
            pltpu.SemaphoreType.DMA,
            pltpu.SemaphoreType.DMA,
        ],
    )


TBLK = 512
NT_BLK = HW // TBLK


def _tr_body(ot_ref, o_ref):
    u = lax.bitcast_convert_type(ot_ref[:, :CH], jnp.uint32)
    lo = lax.bitcast_convert_type((u & jnp.uint32(0xFFFF)).astype(jnp.uint16),
                                  jnp.bfloat16).astype(jnp.float32)
    hi = lax.bitcast_convert_type((u >> 16).astype(jnp.uint16),
                                  jnp.bfloat16).astype(jnp.float32)
    o_ref[...] = jnp.concatenate([lo, hi], axis=1).T


def _stage3_half(outP):
    return pl.pallas_call(
        _tr_body,
        grid=(NT_BLK,),
        in_specs=[pl.BlockSpec((TBLK, CW), lambda j: (j, 0))],
        out_specs=pl.BlockSpec((C, TBLK), lambda j: (0, j)),
        out_shape=jax.ShapeDtypeStruct((C, HW), jnp.float32),
    )(outP)


def kernel(x, Wq, bq, Wk, bk, Wv, bv, Wo, bo):
    xf = x.reshape(B, C, HW)
    halves = []
    for b in range(B):
        halves.append(_stage1_half(xf[b], Wq, bq, Wk, bk, Wv, bv, Wo, bo))
    outs = []
    for b in range(B):
        qt, kt, vt, idx = halves[b]
        outs.append(_sc_attn()(qt, kt, vt, idx))
    res = [_stage3_half(o) for o in outs]
    return jnp.stack(res).reshape(B, C, H, W)


# speedup vs baseline: 1.7494x; 1.1037x over previous
import functools

import jax
import jax.numpy as jnp
from jax import lax
from jax.experimental import pallas as pl
from jax.experimental.pallas import tpu as pltpu
from jax.experimental.pallas import tpu_sc as plsc

B, C, H, W, NREF = 2, 192, 224, 224, 4
HW = H * W
CH = C // 2
CW = 128


BLK = 1024
N_BLK = HW // BLK


def _pack_tc(t):
    lo = lax.bitcast_convert_type(t[:, :CH].astype(jnp.bfloat16), jnp.uint16)
    hi = lax.bitcast_convert_type(t[:, CH:].astype(jnp.bfloat16), jnp.uint16)
    w = (hi.astype(jnp.uint32) << 16) | lo.astype(jnp.uint32)
    w = lax.bitcast_convert_type(w, jnp.float32)
    return jnp.concatenate([w, jnp.zeros((BLK, CW - CH), jnp.float32)], axis=1)


def _proj_body(x_ref, wq_ref, bq_ref, wk_ref, bk_ref, wv_ref, bv_ref,
               wo_ref, bo_ref, qt_ref, kt_ref, vt_ref, idx_ref):
    j = pl.program_id(0)
    xb = x_ref[...]
    xb_bf = xb.astype(jnp.bfloat16)
    cdims = (((0,), (1,)), ((), ()))
    qt = lax.dot_general(xb, wq_ref[...], cdims,
                         preferred_element_type=jnp.float32) + bq_ref[...]
    qt_ref[...] = _pack_tc(qt)
    kt = lax.dot_general(xb_bf, wk_ref[...].astype(jnp.bfloat16), cdims,
                         preferred_element_type=jnp.float32) + bk_ref[...]
    kt_ref[...] = _pack_tc(kt)
    vt = lax.dot_general(xb_bf, wv_ref[...].astype(jnp.bfloat16), cdims,
                         preferred_element_type=jnp.float32) + bv_ref[...]
    vt_ref[...] = _pack_tc(vt)
    offs = lax.dot_general(wo_ref[...], qt, (((1,), (1,)), ((), ())),
                           preferred_element_type=jnp.float32) + bo_ref[...]
    p = j * BLK + lax.broadcasted_iota(jnp.int32, (1, BLK), 1)
    hpos = (p // W).astype(jnp.float32)
    wpos = (p % W).astype(jnp.float32)
    offs = offs.reshape(NREF, 2, BLK)
    ref_w = jnp.clip(wpos + offs[:, 0, :], 0.0, float(W - 1)).astype(jnp.int32)
    ref_h = jnp.clip(hpos + offs[:, 1, :], 0.0, float(H - 1)).astype(jnp.int32)
    idx_ref[...] = ref_h * W + ref_w


def _stage1_half(xf, Wq, bq, Wk, bk, Wv, bv, Wo, bo):
    wspec = pl.BlockSpec((C, C), lambda j: (0, 0))
    rspec = pl.BlockSpec((1, C), lambda j: (0, 0))
    return pl.pallas_call(
        _proj_body,
        grid=(N_BLK,),
        in_specs=[
            pl.BlockSpec((C, BLK), lambda j: (0, j)),
            wspec, rspec,
            wspec, rspec,
            wspec, rspec,
            pl.BlockSpec((2 * NREF, C), lambda j: (0, 0)),
            pl.BlockSpec((2 * NREF, 1), lambda j: (0, 0)),
        ],
        out_specs=[
            pl.BlockSpec((BLK, CW), lambda j: (j, 0)),
            pl.BlockSpec((BLK, CW), lambda j: (j, 0)),
            pl.BlockSpec((BLK, CW), lambda j: (j, 0)),
            pl.BlockSpec((NREF, BLK), lambda j: (0, j)),
        ],
        out_shape=[
            jax.ShapeDtypeStruct((HW, CW), jnp.float32),
            jax.ShapeDtypeStruct((HW, CW), jnp.float32),
            jax.ShapeDtypeStruct((HW, CW), jnp.float32),
            jax.ShapeDtypeStruct((NREF, HW), jnp.int32),
        ],
    )(xf, Wq, bq.reshape(1, C), Wk, bk.reshape(1, C), Wv, bv.reshape(1, C),
      Wo, bo.reshape(2 * NREF, 1))


NC, NS, L = 2, 16, 16
NW = NC * NS
P = 128
SUB = 32
NSUB = P // SUB
NCHUNK = HW // P
NG = CH // L


def _unpack16(w16):
    return plsc.unpack(plsc.bitcast(w16, jnp.bfloat16),
                       format=plsc.PackFormat.INTERLEAVED)


HP = P // 2


def _sc_body(qt_hbm, kt_hbm, vt_hbm, idx_hbm, out_hbm,
             idxv, kg, vg, qv, ov,
             semg0, semg1, semq0, semq1, semo0, semo1, semi):
    wid = lax.axis_index("s") * NC + lax.axis_index("c")
    nchunks = jnp.where(wid < NCHUNK % NW, NCHUNK // NW + 1, NCHUNK // NW)
    semg = (semg0, semg1)
    semq = (semq0, semq1)
    semo = (semo0, semo1)

    def cbase(t):
        return pl.multiple_of((wid + t * NW) * P, P)

    def fire_idx(t):
        return pltpu.async_copy(idx_hbm.at[:, pl.ds(cbase(t), P)],
                                idxv.at[t % 2], semi)

    def fire_q(t, h):
        return pltpu.async_copy(qt_hbm.at[pl.ds(cbase(t) + h * HP, HP)],
                                qv.at[h], semq[h])

    def fire_out(t, h):
        return pltpu.async_copy(ov.at[h],
                                out_hbm.at[pl.ds(cbase(t) + h * HP, HP)],
                                semo[h])

    def fire_g(t, sub):
        slot = sub % 2
        cps = []
        for r in range(NREF):
            ixr = idxv.at[t % 2, r, pl.ds(sub * SUB, SUB)]
            cps.append(pltpu.async_copy(kt_hbm.at[ixr], kg.at[slot, r],
                                        semg[slot]))
            cps.append(pltpu.async_copy(vt_hbm.at[ixr], vg.at[slot, r],
                                        semg[slot]))
        return cps

    def wait_g0():
        for r in range(NREF):
            pltpu.make_async_copy(kt_hbm.at[pl.ds(0, SUB)], kg.at[0, r],
                                  semg[0]).wait()
            pltpu.make_async_copy(kt_hbm.at[pl.ds(0, SUB)], vg.at[0, r],
                                  semg[0]).wait()

    def wait_q(h):
        pltpu.make_async_copy(qt_hbm.at[pl.ds(0, HP)], qv.at[h],
                              semq[h]).wait()

    def wait_out(h):
        pltpu.make_async_copy(ov.at[h], out_hbm.at[pl.ds(0, HP)],
                              semo[h]).wait()

    def compute(sub, slot):
        @plsc.parallel_loop(0, SUB, 1, unroll=2)
        def pos_body(i):
            h = sub // (NSUB // 2)
            pi = (sub % (NSUB // 2)) * SUB + i
            qa, qb = [], []
            for g in range(NG):
                a, b_ = _unpack16(qv[h, pi, pl.ds(g * L, L)])
                qa.append(a)
                qb.append(b_)
            s = []
            for r in range(NREF):
                acc = None
                for g in range(NG):
                    ka, kb = _unpack16(kg[slot, r, i, pl.ds(g * L, L)])
                    term = ka * qa[g] + kb * qb[g]
                    acc = term if acc is None else acc + term
                s.append(jnp.sum(acc))
            for g in range(NG):
                va, vb = _unpack16(vg[slot, 0, i, pl.ds(g * L, L)])
                oa = s[0] * va
                ob = s[0] * vb
                for r in range(1, NREF):
                    va, vb = _unpack16(vg[slot, r, i, pl.ds(g * L, L)])
                    oa = oa + s[r] * va
                    ob = ob + s[r] * vb
                packed = plsc.pack(oa, ob, format=plsc.PackFormat.INTERLEAVED)
                ov[h, pi, pl.ds(g * L, L)] = plsc.bitcast(packed, jnp.float32)

    fire_idx(0).wait()
    fire_g(0, 0)
    fire_q(0, 0)

    def chunk_body(t, _):
        more = t + 1 < nchunks
        fire_q(t, 1)

        @pl.when(more)
        def _():
            fire_idx(t + 1)

        g1 = fire_g(t, 1)
        wait_g0()
        wait_q(0)

        @pl.when(t > 0)
        def _():
            wait_out(0)

        compute(0, 0)

        g2 = fire_g(t, 2)
        for cp in g1:
            cp.wait()
        compute(1, 1)
        fire_out(t, 0)

        @pl.when(more)
        def _():
            fire_q(t + 1, 0)

        g3 = fire_g(t, 3)
        for cp in g2:
            cp.wait()
        wait_q(1)

        @pl.when(t > 0)
        def _():
            wait_out(1)

        compute(2, 0)

        @pl.when(more)
        def _():
            pltpu.make_async_copy(idx_hbm.at[:, pl.ds(0, P)],
                                  idxv.at[(t + 1) % 2], semi).wait()
            fire_g(t + 1, 0)

        for cp in g3:
            cp.wait()
        compute(3, 1)
        fire_out(t, 1)
        return ()

    lax.fori_loop(0, nchunks, chunk_body, (), unroll=False)
    wait_out(0)
    wait_out(1)


@functools.cache
def _sc_attn():
    return pl.kernel(
        _sc_body,
        out_type=jax.ShapeDtypeStruct((HW, CW), jnp.float32),
        mesh=plsc.VectorSubcoreMesh(core_axis_name="c", subcore_axis_name="s",
                                    num_cores=NC, num_subcores=NS),
        compiler_params=pltpu.CompilerParams(needs_layout_passes=False),
        scratch_types=[
            pltpu.VMEM((2, NREF, P), jnp.int32),
            pltpu.VMEM((2, NREF, SUB, CW), jnp.float32),
            pltpu.VMEM((2, NREF, SUB, CW), jnp.float32),
            pltpu.VMEM((2, HP, CW), jnp.float32),
            pltpu.VMEM((2, HP, CW), jnp.float32),
            pltpu.SemaphoreType.DMA,
            pltpu.SemaphoreType.DMA,
            pltpu.SemaphoreType.DMA,
            pltpu.SemaphoreType.DMA,
            pltpu.SemaphoreType.DMA,
            pltpu.SemaphoreType.DMA,
            pltpu.SemaphoreType.DMA,
        ],
    )


TBLK = 512
NT_BLK = HW // TBLK


def _tr_body(ot_ref, o_ref):
    u = lax.bitcast_convert_type(ot_ref[:, :CH], jnp.uint32)
    lo = lax.bitcast_convert_type((u & jnp.uint32(0xFFFF)).astype(jnp.uint16),
                                  jnp.bfloat16).astype(jnp.float32)
    hi = lax.bitcast_convert_type((u >> 16).astype(jnp.uint16),
                                  jnp.bfloat16).astype(jnp.float32)
    o_ref[...] = jnp.concatenate([lo, hi], axis=1).T


def _stage3_half(outP):
    return pl.pallas_call(
        _tr_body,
        grid=(NT_BLK,),
        in_specs=[pl.BlockSpec((TBLK, CW), lambda j: (j, 0))],
        out_specs=pl.BlockSpec((C, TBLK), lambda j: (0, j)),
        out_shape=jax.ShapeDtypeStruct((C, HW), jnp.float32),
    )(outP)


def kernel(x, Wq, bq, Wk, bk, Wv, bv, Wo, bo):
    xf = x.reshape(B, C, HW)
    halves = []
    for b in range(B):
        halves.append(_stage1_half(xf[b], Wq, bq, Wk, bk, Wv, bv, Wo, bo))
    outs = []
    for b in range(B):
        qt, kt, vt, idx = halves[b]
        outs.append(_sc_attn()(qt, kt, vt, idx))
    res = [_stage3_half(o) for o in outs]
    return jnp.stack(res).reshape(B, C, H, W)
